# baseline (speedup 1.0000x reference)
import jax
import jax.numpy as jnp
from jax.experimental import pallas as pl
from jax.experimental.pallas import tpu as pltpu

_N = 4096
_D = 64
_STEPS = 2
_BR = 512
_NB = _N // _BR


def _body(adj_ref, x_ref, nn_ref, wcat_ref, bcat_ref,
          out_ref, h_ref, b0_ref, b1_ref, adjc_ref):
    s = pl.program_id(0)
    i = pl.program_id(1)
    row0 = i * _BR
    d = lambda p, q: jnp.dot(p, q, preferred_element_type=jnp.float32)

    @pl.when((s == 0) & (i == 0))
    def _():
        b0_ref[...] = x_ref[...].astype(jnp.bfloat16)

    def run(hf_ref, hb_ref, write_out):
        if write_out:
            ab = adjc_ref[pl.ds(row0, _BR), :]
        else:
            ab = adj_ref[...].astype(jnp.bfloat16)
            adjc_ref[pl.ds(row0, _BR), :] = ab
        u = d(ab, hb_ref[...])
        u = u / (nn_ref[...] + 1e-7)
        h_rows = hf_ref[pl.ds(row0, _BR), :]
        hb_rows = hb_ref[pl.ds(row0, _BR), :]
        g = d(jnp.concatenate([u.astype(jnp.bfloat16), hb_rows], axis=1),
              wcat_ref[...]) + bcat_ref[...]
        r = jax.nn.sigmoid(g[:, :_D])
        z = jax.nn.sigmoid(g[:, _D:2 * _D])
        n = jnp.tanh(g[:, 2 * _D:3 * _D] + r * g[:, 3 * _D:])
        h_new = (1.0 - z) * n + z * h_rows
        if write_out:
            out_ref[:, :_D] = x_ref[pl.ds(row0, _BR), :]
            out_ref[:, _D:] = h_new
        else:
            h_ref[pl.ds(row0, _BR), :] = h_new
            b1_ref[pl.ds(row0, _BR), :] = h_new.astype(jnp.bfloat16)

    @pl.when(s == 0)
    def _():
        run(x_ref, b0_ref, False)

    @pl.when(s == 1)
    def _():
        run(h_ref, b1_ref, True)


def kernel(x, adj_matrix, num_neighbors, W_ih, W_hh, b_ih, b_hh):
    wi = W_ih.T
    wh = W_hh.T
    zz = jnp.zeros((_D, _D), jnp.float32)
    wcat = jnp.block([
        [wi[:, :_D], wi[:, _D:2 * _D], wi[:, 2 * _D:], zz],
        [wh[:, :_D], wh[:, _D:2 * _D], zz, wh[:, 2 * _D:]],
    ]).astype(jnp.bfloat16)
    bcat = jnp.concatenate([
        b_ih[:_D] + b_hh[:_D],
        b_ih[_D:2 * _D] + b_hh[_D:2 * _D],
        b_ih[2 * _D:],
        b_hh[2 * _D:],
    ]).reshape(1, 4 * _D)
    nn = num_neighbors.reshape(_N, 1)
    const = lambda s, i: (0, 0)
    rows = lambda s, i: (i, 0)
    return pl.pallas_call(
        _body,
        grid=(_STEPS, _NB),
        in_specs=[
            pl.BlockSpec((_BR, _N), lambda s, i: (i * (1 - s), 0)),
            pl.BlockSpec((_N, _D), const),
            pl.BlockSpec((_BR, 1), rows),
            pl.BlockSpec((2 * _D, 4 * _D), const),
            pl.BlockSpec((1, 4 * _D), const),
        ],
        out_specs=pl.BlockSpec((_BR, 2 * _D), rows),
        out_shape=jax.ShapeDtypeStruct((_N, 2 * _D), jnp.float32),
        scratch_shapes=[
            pltpu.VMEM((_N, _D), jnp.float32),
            pltpu.VMEM((_N, _D), jnp.bfloat16),
            pltpu.VMEM((_N, _D), jnp.bfloat16),
            pltpu.VMEM((_N, _N), jnp.bfloat16),
        ],
    )(adj_matrix, x, nn, wcat, bcat)


# speedup vs baseline: 1.0017x; 1.0017x over previous
import jax
import jax.numpy as jnp
from jax.experimental import pallas as pl
from jax.experimental.pallas import tpu as pltpu

_N = 4096
_D = 64
_STEPS = 2
_BR = 512
_NB = _N // _BR


def _body(adj_ref, x_ref, nn_ref, wcat_ref, bcat_ref,
          out_ref, h_ref, b0_ref, b1_ref, adjc_ref):
    s = pl.program_id(0)
    i = pl.program_id(1)
    row0 = i * _BR
    d = lambda p, q: jnp.dot(p, q, preferred_element_type=jnp.float32)

    @pl.when((s == 0) & (i == 0))
    def _():
        b0_ref[...] = x_ref[...].astype(jnp.bfloat16)

    _HB = _BR // 2

    def run(hf_ref, hb_ref, write_out):
        for half in range(2):
            r0 = row0 + half * _HB
            if write_out:
                ab = adjc_ref[pl.ds(r0, _HB), :]
            else:
                ab = adj_ref[pl.ds(half * _HB, _HB), :].astype(jnp.bfloat16)
                adjc_ref[pl.ds(r0, _HB), :] = ab
            u = d(ab, hb_ref[...])
            u = u / (nn_ref[pl.ds(half * _HB, _HB), :] + 1e-7)
            h_rows = hf_ref[pl.ds(r0, _HB), :]
            hb_rows = hb_ref[pl.ds(r0, _HB), :]
            g = d(jnp.concatenate([u.astype(jnp.bfloat16), hb_rows], axis=1),
                  wcat_ref[...]) + bcat_ref[...]
            r = jax.nn.sigmoid(g[:, :_D])
            z = jax.nn.sigmoid(g[:, _D:2 * _D])
            n = jnp.tanh(g[:, 2 * _D:3 * _D] + r * g[:, 3 * _D:])
            h_new = (1.0 - z) * n + z * h_rows
            if write_out:
                out_ref[pl.ds(half * _HB, _HB), :_D] = x_ref[pl.ds(r0, _HB), :]
                out_ref[pl.ds(half * _HB, _HB), _D:] = h_new
            else:
                h_ref[pl.ds(r0, _HB), :] = h_new
                b1_ref[pl.ds(r0, _HB), :] = h_new.astype(jnp.bfloat16)

    @pl.when(s == 0)
    def _():
        run(x_ref, b0_ref, False)

    @pl.when(s == 1)
    def _():
        run(h_ref, b1_ref, True)


def kernel(x, adj_matrix, num_neighbors, W_ih, W_hh, b_ih, b_hh):
    wi = W_ih.T
    wh = W_hh.T
    zz = jnp.zeros((_D, _D), jnp.float32)
    wcat = jnp.block([
        [wi[:, :_D], wi[:, _D:2 * _D], wi[:, 2 * _D:], zz],
        [wh[:, :_D], wh[:, _D:2 * _D], zz, wh[:, 2 * _D:]],
    ]).astype(jnp.bfloat16)
    bcat = jnp.concatenate([
        b_ih[:_D] + b_hh[:_D],
        b_ih[_D:2 * _D] + b_hh[_D:2 * _D],
        b_ih[2 * _D:],
        b_hh[2 * _D:],
    ]).reshape(1, 4 * _D)
    nn = num_neighbors.reshape(_N, 1)
    const = lambda s, i: (0, 0)
    rows = lambda s, i: (i, 0)
    return pl.pallas_call(
        _body,
        grid=(_STEPS, _NB),
        in_specs=[
            pl.BlockSpec((_BR, _N), lambda s, i: (i * (1 - s), 0)),
            pl.BlockSpec((_N, _D), const),
            pl.BlockSpec((_BR, 1), rows),
            pl.BlockSpec((2 * _D, 4 * _D), const),
            pl.BlockSpec((1, 4 * _D), const),
        ],
        out_specs=pl.BlockSpec((_BR, 2 * _D), rows),
        out_shape=jax.ShapeDtypeStruct((_N, 2 * _D), jnp.float32),
        scratch_shapes=[
            pltpu.VMEM((_N, _D), jnp.float32),
            pltpu.VMEM((_N, _D), jnp.bfloat16),
            pltpu.VMEM((_N, _D), jnp.bfloat16),
            pltpu.VMEM((_N, _N), jnp.bfloat16),
        ],
    )(adj_matrix, x, nn, wcat, bcat)
